# baseline (device time: 19189 ns/iter reference)
import jax
import jax.numpy as jnp
from jax import lax
from jax.experimental import pallas as pl
from jax.experimental.pallas import tpu as pltpu

M = 1024
N_PER = 512
C = 8
R = M // C


def kernel(x):
    def body(
        x_hbm,
        out_ref,
        xstage,
        xmine,
        xsend,
        xrecv,
        stage_sems,
        mine_sem,
        xsend_sems,
        xrecv_sems,
    ):
        my_x = lax.axis_index("x")
        my_y = lax.axis_index("y")
        my_z = lax.axis_index("z")
        xpeer = (1 - my_x, my_y, my_z)

        col_send = (1 - my_x) * N_PER
        col_mine = my_x * N_PER

        mine_copy = pltpu.make_async_copy(
            x_hbm.at[0, :, pl.ds(col_mine, N_PER)], xmine, mine_sem
        )
        mine_copy.start()
        stage_copies = []
        for c in range(C):
            cp = pltpu.make_async_copy(
                x_hbm.at[0, pl.ds(c * R, R), pl.ds(col_send, N_PER)],
                xstage.at[c],
                stage_sems.at[c],
            )
            cp.start()
            stage_copies.append(cp)

        barrier = pltpu.get_barrier_semaphore()
        pl.semaphore_signal(
            barrier, inc=1, device_id=xpeer, device_id_type=pl.DeviceIdType.MESH
        )
        pl.semaphore_wait(barrier, 1)

        xrdmas = []
        for c in range(C):
            stage_copies[c].wait()
            xsend[c] = xstage[c].astype(jnp.bfloat16)
            r = pltpu.make_async_remote_copy(
                src_ref=xsend.at[c],
                dst_ref=xrecv.at[c],
                send_sem=xsend_sems.at[c],
                recv_sem=xrecv_sems.at[c],
                device_id=xpeer,
                device_id_type=pl.DeviceIdType.MESH,
            )
            r.start()
            xrdmas.append(r)

        mine_copy.wait()

        for c in range(C):
            xrdmas[c].wait_recv()
            out_ref[c * R : (c + 1) * R, :] = xmine[
                c * R : (c + 1) * R, :
            ] + xrecv[c].astype(jnp.float32)

        for c in range(C):
            xrdmas[c].wait_send()

    return pl.pallas_call(
        body,
        out_shape=jax.ShapeDtypeStruct((M, N_PER), jnp.float32),
        in_specs=[pl.BlockSpec(memory_space=pl.ANY)],
        out_specs=pl.BlockSpec(memory_space=pltpu.VMEM),
        scratch_shapes=[
            pltpu.VMEM((C, R, N_PER), jnp.float32),
            pltpu.VMEM((M, N_PER), jnp.float32),
            pltpu.VMEM((C, R, N_PER), jnp.bfloat16),
            pltpu.VMEM((C, R, N_PER), jnp.bfloat16),
            pltpu.SemaphoreType.DMA((C,)),
            pltpu.SemaphoreType.DMA,
            pltpu.SemaphoreType.DMA((C,)),
            pltpu.SemaphoreType.DMA((C,)),
        ],
        compiler_params=pltpu.CompilerParams(collective_id=0),
    )(x)
